# Initial kernel scaffold; baseline (speedup 1.0000x reference)
import jax
import jax.numpy as jnp
from jax.experimental import pallas as pl


def kernel(x, edge_index, W1, b1, W2, b2, Wd1, bd1, Wd2, bd2):
    raise NotImplementedError("write your pallas kernel here")


# speedup vs baseline: 27.0488x; 27.0488x over previous
import functools

import jax
import jax.numpy as jnp
from jax import lax
from jax.experimental import pallas as pl
from jax.experimental.pallas import tpu as pltpu
from jax.experimental.pallas import tpu_sc as plsc

NC = 2
NS = 16
NW = NC * NS
LK = 128
DW = 128

@functools.lru_cache(maxsize=None)
def _mesh():
    return plsc.VectorSubcoreMesh(core_axis_name="c", subcore_axis_name="s",
                                  num_cores=NC, num_subcores=NS)


def _deg_body(N, NR, CH, dst_hbm, zeros_hbm, ones_hbm, out_hbm,
              idx_v, ones_v, acc_sh, sem):
    c = lax.axis_index("c")
    s = lax.axis_index("s")
    w = c * NS + s
    zr = NR // NS
    pltpu.sync_copy(zeros_hbm.at[pl.ds(s * zr, zr), :],
                    acc_sh.at[pl.ds(s * zr, zr), :])
    pltpu.sync_copy(ones_hbm, ones_v)
    pltpu.sync_copy(dst_hbm.at[w], idx_v)
    plsc.subcore_barrier()

    K = 8
    for k in range(K):
        pltpu.async_copy(ones_v, acc_sh.at[idx_v.at[k]], sem, add=True)

    def body(j, carry):
        pltpu.make_async_copy(ones_v, acc_sh.at[idx_v.at[j]], sem).wait()

        @pl.when(j + K < CH)
        def _():
            pltpu.async_copy(ones_v, acc_sh.at[idx_v.at[j + K]], sem,
                             add=True)
        return carry

    lax.fori_loop(0, CH, body, 0)
    plsc.subcore_barrier()
    pltpu.sync_copy(acc_sh.at[pl.ds(s * zr, zr), :],
                    out_hbm.at[c].at[pl.ds(s * zr, zr), :])


def _scat_body(N, D, NR, CH, NB, h_hbm, src_hbm, dst_hbm, zeros_hbm, out_hbm,
               src_v, dst_v, rows_v, acc_sh, *sems):
    c = lax.axis_index("c")
    s = lax.axis_index("s")
    w = c * NS + s
    zr = NR // NS

    CHH = CH // 2
    for half in range(2):
        pltpu.sync_copy(src_hbm.at[w, pl.ds(half * CHH, CHH)], src_v)
        pltpu.sync_copy(dst_hbm.at[w, pl.ds(half * CHH, CHH)], dst_v)

        for b in range(NB):
            pltpu.async_copy(h_hbm.at[src_v.at[b]], rows_v.at[b], sems[b])

        if half == 0:
            pltpu.sync_copy(zeros_hbm.at[pl.ds(s * zr, zr), :],
                            acc_sh.at[pl.ds(s * zr, zr), :])
            plsc.subcore_barrier()

        def body(g, carry):
            for b in range(NB):
                j = g * NB + b
                pltpu.make_async_copy(h_hbm.at[src_v.at[j]], rows_v.at[b],
                                      sems[b]).wait()
                pltpu.sync_copy(rows_v.at[b], acc_sh.at[dst_v.at[j]],
                                add=True)

                @pl.when(j + NB < CHH)
                def _():
                    pltpu.async_copy(h_hbm.at[src_v.at[j + NB]],
                                     rows_v.at[b], sems[b])
            return carry

        lax.fori_loop(0, CHH // NB, body, 0)
    plsc.subcore_barrier()
    pltpu.sync_copy(acc_sh.at[pl.ds(s * zr, zr), :],
                    out_hbm.at[c].at[pl.ds(s * zr, zr), :])


@functools.lru_cache(maxsize=None)
def _make_deg(N, NR, CH):
    return pl.kernel(
        functools.partial(_deg_body, N, NR, CH),
        out_type=jax.ShapeDtypeStruct((NC, NR, DW), jnp.float32),
        mesh=_mesh(),
        scratch_types=[
            pltpu.VMEM((CH, LK), jnp.int32),
            pltpu.VMEM((LK, DW), jnp.float32),
            pltpu.VMEM_SHARED((NR, DW), jnp.float32),
            pltpu.SemaphoreType.DMA,
        ],
    )


@functools.lru_cache(maxsize=None)
def _make_scat(N, D, NR, CH, NB=2):
    return pl.kernel(
        functools.partial(_scat_body, N, D, NR, CH, NB),
        out_type=jax.ShapeDtypeStruct((NC, NR, D), jnp.float32),
        mesh=_mesh(),
        scratch_types=[
            pltpu.VMEM((CH // 2, LK), jnp.int32),
            pltpu.VMEM((CH // 2, LK), jnp.int32),
            pltpu.VMEM((NB, LK, D), jnp.float32),
            pltpu.VMEM_SHARED((NR, D), jnp.float32),
        ] + [pltpu.SemaphoreType.DMA] * NB,
    )


def _dotT(a, w):
    return lax.dot_general(a, w, (((1,), (1,)), ((), ())),
                           preferred_element_type=jnp.float32,
                           precision=lax.Precision.DEFAULT)


def _mm_body(x_ref, w_ref, o_ref):
    o_ref[...] = _dotT(x_ref[...], w_ref[...])


def _scale_body(m_ref, deg_ref, o_ref):
    o_ref[...] = m_ref[...] * lax.rsqrt(deg_ref[...])


def _combine_mm_body(pa_ref, pb_ref, h_ref, deg_ref, b_ref, w_ref, o_ref):
    dinv = lax.rsqrt(deg_ref[...])
    h = (pa_ref[0] + pb_ref[0] + h_ref[...]) * dinv + b_ref[...]
    h = jnp.maximum(h, 0.0)
    o_ref[...] = _dotT(h, w_ref[...]) * dinv


def _decode_body(pa_ref, pb_ref, h_ref, deg_ref, b2_ref,
                 wd1_ref, bd1_ref, wd2_ref, bd2_ref, o_ref):
    dinv = lax.rsqrt(deg_ref[...])
    z = (pa_ref[0] + pb_ref[0] + h_ref[...]) * dinv + b2_ref[...]
    d = jnp.maximum(_dotT(z, wd1_ref[...]) + bd1_ref[...], 0.0)
    o_ref[...] = _dotT(d, wd2_ref[...]) + bd2_ref[...]


def _row_specs(R, ncols, n):
    return [pl.BlockSpec((R, ncols), lambda i: (i, 0)) for _ in range(n)]


def _part_specs(R, ncols):
    return [pl.BlockSpec((1, R, ncols), lambda i, c=c: (c, i, 0))
            for c in range(NC)]


def _full_spec(shape):
    return pl.BlockSpec(shape, lambda i: tuple(0 for _ in shape))


def _tc_call(body, N, R, in_specs, out_cols, *args):
    return pl.pallas_call(
        body,
        grid=(N // R,),
        in_specs=in_specs,
        out_specs=pl.BlockSpec((R, out_cols), lambda i: (i, 0)),
        out_shape=jax.ShapeDtypeStruct((N, out_cols), jnp.float32),
    )(*args)


def kernel(x, edge_index, W1, b1, W2, b2, Wd1, bd1, Wd2, bd2):
    N, Din = x.shape
    E = edge_index.shape[1]
    H = W1.shape[0]
    O = Wd2.shape[0]

    CH = -(-(-(-E // (NW * LK))) // 16) * 16
    EP = NW * CH * LK
    NR = -(-(N + 1) // 128) * 128

    src = edge_index[0]
    dst = edge_index[1]
    if EP > E:
        pad = jnp.arange(EP - E, dtype=edge_index.dtype)
        src = jnp.concatenate([src, pad % N])
        dst = jnp.concatenate([dst, N + pad % (NR - N)])
    src3 = src.reshape(NW, CH, LK).astype(jnp.int32)
    dst3 = dst.reshape(NW, CH, LK).astype(jnp.int32)

    ones_deg = jnp.ones((LK, DW), jnp.float32)
    zeros_acc = jnp.zeros((NR, H), jnp.float32)

    R = 1000

    degp = _make_deg(N, NR, CH)(dst3, zeros_acc, ones_deg)
    m1 = _tc_call(_mm_body, N, R,
                  _row_specs(R, Din, 1) + [_full_spec(W1.shape)], H, x, W1)
    degsum = degp[0, :N, 0:1] + degp[1, :N, 0:1] + 1.0
    b1r = b1.reshape(1, H)
    b2r = b2.reshape(1, W2.shape[0])
    bd1r = bd1.reshape(1, Wd1.shape[0])
    bd2r = bd2.reshape(1, O)

    h1p = _tc_call(_scale_body, N, R,
                   _row_specs(R, H, 1) + _row_specs(R, 1, 1), H, m1, degsum)

    p1 = _make_scat(N, H, NR, CH)(h1p, src3, dst3, zeros_acc)

    h2p = _tc_call(
        _combine_mm_body, N, R,
        _part_specs(R, H) + _row_specs(R, H, 1) + _row_specs(R, 1, 1)
        + [_full_spec((1, H)), _full_spec(W2.shape)],
        W2.shape[0], p1, p1, h1p, degsum, b1r, W2)

    p2 = _make_scat(N, W2.shape[0], NR, CH)(h2p, src3, dst3, zeros_acc)

    x_rec = _tc_call(
        _decode_body, N, R,
        _part_specs(R, W2.shape[0]) + _row_specs(R, W2.shape[0], 1)
        + _row_specs(R, 1, 1)
        + [_full_spec((1, W2.shape[0])), _full_spec(Wd1.shape),
           _full_spec((1, Wd1.shape[0])), _full_spec(Wd2.shape),
           _full_spec((1, O))],
        O, p2, p2, h2p, degsum, b2r, Wd1, bd1r, Wd2, bd2r)

    return x_rec


# speedup vs baseline: 27.6791x; 1.0233x over previous
import functools

import jax
import jax.numpy as jnp
from jax import lax
from jax.experimental import pallas as pl
from jax.experimental.pallas import tpu as pltpu
from jax.experimental.pallas import tpu_sc as plsc

NC = 2
NS = 16
NW = NC * NS
LK = 128
DW = 128

@functools.lru_cache(maxsize=None)
def _mesh():
    return plsc.VectorSubcoreMesh(core_axis_name="c", subcore_axis_name="s",
                                  num_cores=NC, num_subcores=NS)


def _deg_body(N, NR, CH, dst_hbm, zeros_hbm, ones_hbm, out_hbm,
              idx_v, ones_v, acc_sh, sem):
    c = lax.axis_index("c")
    s = lax.axis_index("s")
    w = c * NS + s
    zr = NR // NS
    pltpu.sync_copy(zeros_hbm.at[pl.ds(s * zr, zr), :],
                    acc_sh.at[pl.ds(s * zr, zr), :])
    pltpu.sync_copy(ones_hbm, ones_v)
    pltpu.sync_copy(dst_hbm.at[w], idx_v)
    plsc.subcore_barrier()

    K = 8
    for k in range(K):
        pltpu.async_copy(ones_v, acc_sh.at[idx_v.at[k]], sem, add=True)

    def body(j, carry):
        pltpu.make_async_copy(ones_v, acc_sh.at[idx_v.at[j]], sem).wait()

        @pl.when(j + K < CH)
        def _():
            pltpu.async_copy(ones_v, acc_sh.at[idx_v.at[j + K]], sem,
                             add=True)
        return carry

    lax.fori_loop(0, CH, body, 0)
    plsc.subcore_barrier()
    pltpu.sync_copy(acc_sh.at[pl.ds(s * zr, zr), :],
                    out_hbm.at[c].at[pl.ds(s * zr, zr), :])


def _scat_body(N, D, NR, CH, NB, h_hbm, src_hbm, dst_hbm, zeros_hbm, out_hbm,
               src_v, dst_v, rows_v, acc_sh, *sems):
    c = lax.axis_index("c")
    s = lax.axis_index("s")
    w = c * NS + s
    zr = NR // NS

    CHH = CH // 2
    for half in range(2):
        pltpu.sync_copy(src_hbm.at[w, pl.ds(half * CHH, CHH)], src_v)
        pltpu.sync_copy(dst_hbm.at[w, pl.ds(half * CHH, CHH)], dst_v)

        for b in range(NB):
            pltpu.async_copy(h_hbm.at[src_v.at[b]], rows_v.at[b], sems[b])

        if half == 0:
            pltpu.sync_copy(zeros_hbm.at[pl.ds(s * zr, zr), :],
                            acc_sh.at[pl.ds(s * zr, zr), :])
            plsc.subcore_barrier()

        def body(g, carry):
            for b in range(NB):
                j = g * NB + b
                pltpu.make_async_copy(h_hbm.at[src_v.at[j]], rows_v.at[b],
                                      sems[b]).wait()
                pltpu.sync_copy(rows_v.at[b], acc_sh.at[dst_v.at[j]],
                                add=True)

                @pl.when(j + NB < CHH)
                def _():
                    pltpu.async_copy(h_hbm.at[src_v.at[j + NB]],
                                     rows_v.at[b], sems[b])
            return carry

        lax.fori_loop(0, CHH // NB, body, 0)
    plsc.subcore_barrier()
    pltpu.sync_copy(acc_sh.at[pl.ds(s * zr, zr), :],
                    out_hbm.at[c].at[pl.ds(s * zr, zr), :])


@functools.lru_cache(maxsize=None)
def _make_deg(N, NR, CH):
    return pl.kernel(
        functools.partial(_deg_body, N, NR, CH),
        out_type=jax.ShapeDtypeStruct((NC, NR, DW), jnp.float32),
        mesh=_mesh(),
        scratch_types=[
            pltpu.VMEM((CH, LK), jnp.int32),
            pltpu.VMEM((LK, DW), jnp.float32),
            pltpu.VMEM_SHARED((NR, DW), jnp.float32),
            pltpu.SemaphoreType.DMA,
        ],
    )


@functools.lru_cache(maxsize=None)
def _make_scat(N, D, NR, CH, NB=2):
    return pl.kernel(
        functools.partial(_scat_body, N, D, NR, CH, NB),
        out_type=jax.ShapeDtypeStruct((NC, NR, D), jnp.float32),
        mesh=_mesh(),
        scratch_types=[
            pltpu.VMEM((CH // 2, LK), jnp.int32),
            pltpu.VMEM((CH // 2, LK), jnp.int32),
            pltpu.VMEM((NB, LK, D), jnp.float32),
            pltpu.VMEM_SHARED((NR, D), jnp.float32),
        ] + [pltpu.SemaphoreType.DMA] * NB,
    )


def _dotT(a, w):
    return lax.dot_general(a, w, (((1,), (1,)), ((), ())),
                           preferred_element_type=jnp.float32,
                           precision=lax.Precision.DEFAULT)


def _mm_body(x_ref, w_ref, o_ref):
    o_ref[...] = _dotT(x_ref[...], w_ref[...])


def _scale_body(m_ref, deg_ref, o_ref):
    o_ref[...] = m_ref[...] * lax.rsqrt(deg_ref[...])


def _combine_mm_body(pa_ref, pb_ref, h_ref, deg_ref, b_ref, w_ref, o_ref):
    dinv = lax.rsqrt(deg_ref[...])
    h = (pa_ref[0] + pb_ref[0] + h_ref[...]) * dinv + b_ref[...]
    h = jnp.maximum(h, 0.0)
    o_ref[...] = _dotT(h, w_ref[...]) * dinv


def _decode_body(pa_ref, pb_ref, h_ref, deg_ref, b2_ref,
                 wd1_ref, bd1_ref, wd2_ref, bd2_ref, o_ref):
    dinv = lax.rsqrt(deg_ref[...])
    z = (pa_ref[0] + pb_ref[0] + h_ref[...]) * dinv + b2_ref[...]
    d = jnp.maximum(_dotT(z, wd1_ref[...]) + bd1_ref[...], 0.0)
    o_ref[...] = _dotT(d, wd2_ref[...]) + bd2_ref[...]


def _row_specs(R, ncols, n):
    return [pl.BlockSpec((R, ncols), lambda i: (i, 0)) for _ in range(n)]


def _part_specs(R, ncols):
    return [pl.BlockSpec((1, R, ncols), lambda i, c=c: (c, i, 0))
            for c in range(NC)]


def _full_spec(shape):
    return pl.BlockSpec(shape, lambda i: tuple(0 for _ in shape))


def _tc_call(body, N, R, in_specs, out_cols, *args):
    return pl.pallas_call(
        body,
        grid=(N // R,),
        in_specs=in_specs,
        out_specs=pl.BlockSpec((R, out_cols), lambda i: (i, 0)),
        out_shape=jax.ShapeDtypeStruct((N, out_cols), jnp.float32),
    )(*args)


def kernel(x, edge_index, W1, b1, W2, b2, Wd1, bd1, Wd2, bd2):
    N, Din = x.shape
    E = edge_index.shape[1]
    H = W1.shape[0]
    O = Wd2.shape[0]

    CH = -(-(-(-E // (NW * LK))) // 16) * 16
    EP = NW * CH * LK
    NR = -(-(N + 1) // 128) * 128

    src = edge_index[0]
    dst = edge_index[1]
    if EP > E:
        pad = jnp.arange(EP - E, dtype=edge_index.dtype)
        src = jnp.concatenate([src, pad % N])
        dst = jnp.concatenate([dst, N + pad % (NR - N)])
    src3 = src.reshape(NW, CH, LK).astype(jnp.int32)
    dst3 = dst.reshape(NW, CH, LK).astype(jnp.int32)

    ones_deg = jnp.ones((LK, DW), jnp.float32)
    zeros_acc = jnp.zeros((NR, H), jnp.float32)

    R = 2000

    degp = _make_deg(N, NR, CH)(dst3, zeros_acc, ones_deg)
    m1 = _tc_call(_mm_body, N, R,
                  _row_specs(R, Din, 1) + [_full_spec(W1.shape)], H, x, W1)
    degsum = degp[0, :N, 0:1] + degp[1, :N, 0:1] + 1.0
    b1r = b1.reshape(1, H)
    b2r = b2.reshape(1, W2.shape[0])
    bd1r = bd1.reshape(1, Wd1.shape[0])
    bd2r = bd2.reshape(1, O)

    h1p = _tc_call(_scale_body, N, R,
                   _row_specs(R, H, 1) + _row_specs(R, 1, 1), H, m1, degsum)

    p1 = _make_scat(N, H, NR, CH)(h1p, src3, dst3, zeros_acc)

    h2p = _tc_call(
        _combine_mm_body, N, R,
        _part_specs(R, H) + _row_specs(R, H, 1) + _row_specs(R, 1, 1)
        + [_full_spec((1, H)), _full_spec(W2.shape)],
        W2.shape[0], p1, p1, h1p, degsum, b1r, W2)

    p2 = _make_scat(N, W2.shape[0], NR, CH)(h2p, src3, dst3, zeros_acc)

    x_rec = _tc_call(
        _decode_body, N, R,
        _part_specs(R, W2.shape[0]) + _row_specs(R, W2.shape[0], 1)
        + _row_specs(R, 1, 1)
        + [_full_spec((1, W2.shape[0])), _full_spec(Wd1.shape),
           _full_spec((1, Wd1.shape[0])), _full_spec(Wd2.shape),
           _full_spec((1, O))],
        O, p2, p2, h2p, degsum, b2r, Wd1, bd1r, Wd2, bd2r)

    return x_rec
